# baseline (speedup 1.0000x reference)
import jax
import jax.numpy as jnp
from jax import lax
from jax.experimental import pallas as pl
from jax.experimental.pallas import tpu as pltpu
from jax.experimental.pallas import tpu_sc as plsc

B, H, W, C = 4, 224, 224, 96
N = B * H * W
NC, NS = 2, 16
NW = NC * NS
RPW = N // NW
R = 448
NB = RPW // R
Bb = R * C
L = 16
NJ = C // L


def _permute_body(x_hbm, perm_hbm, out_hbm, perm_v, in_v, out_v):
    wid = lax.axis_index("s") * NC + lax.axis_index("c")
    pltpu.sync_copy(perm_hbm, perm_v)
    pj = [perm_v[pl.ds(L * j, L)] for j in range(NJ)]
    wbase = wid * (RPW * C)

    @pl.loop(0, NB)
    def _blocks(b):
        base = wbase + b * Bb
        pltpu.sync_copy(x_hbm.at[pl.ds(base, Bb)], in_v)

        @plsc.parallel_loop(0, R, unroll=4)
        def _rows(r):
            ro = r * C
            for j in range(NJ):
                v = plsc.load_gather(in_v, [pj[j] + ro])
                out_v[pl.ds(ro + L * j, L)] = v

        pltpu.sync_copy(out_v, out_hbm.at[pl.ds(base, Bb)])


@jax.jit
def _permute(x_flat, permutation):
    return pl.kernel(
        _permute_body,
        out_type=jax.ShapeDtypeStruct((N * C,), jnp.float32),
        mesh=plsc.VectorSubcoreMesh(core_axis_name="c", subcore_axis_name="s"),
        compiler_params=pltpu.CompilerParams(needs_layout_passes=False),
        scratch_types=[
            pltpu.VMEM((C,), jnp.int32),
            pltpu.VMEM((Bb,), jnp.float32),
            pltpu.VMEM((Bb,), jnp.float32),
        ],
    )(x_flat, permutation)


def kernel(x, permutation):
    out = _permute(x.reshape(-1), permutation)
    return out.reshape(x.shape)


# speedup vs baseline: 2.8639x; 2.8639x over previous
import jax
import jax.numpy as jnp
from jax import lax
from jax.experimental import pallas as pl
from jax.experimental.pallas import tpu as pltpu
from jax.experimental.pallas import tpu_sc as plsc

B, H, W, C = 4, 224, 224, 96
G = B * H
NC, NS = 2, 16
NW = NC * NS
GPW = G // NW
L = 16
NK = W // L
NJ = C // L


def _permute_body(x_hbm, perm_hbm, out_hbm, perm_v, in_v, out_v):
    wid = lax.axis_index("s") * NC + lax.axis_index("c")
    pltpu.sync_copy(perm_hbm, perm_v)
    pj = [perm_v[pl.ds(L * j, L)] for j in range(NJ)]
    srcs = [pj[j][l] for j in range(NJ) for l in range(L)]

    @pl.loop(0, GPW)
    def _groups(s):
        g = wid * GPW + s
        pltpu.sync_copy(x_hbm.at[g], in_v)
        for c in range(C):
            src = srcs[c]
            for k in range(NK):
                out_v[c, pl.ds(L * k, L)] = in_v[src, pl.ds(L * k, L)]
        pltpu.sync_copy(out_v, out_hbm.at[g])


@jax.jit
def _permute(x3, permutation):
    return pl.kernel(
        _permute_body,
        out_type=jax.ShapeDtypeStruct((G, C, W), jnp.float32),
        mesh=plsc.VectorSubcoreMesh(core_axis_name="c", subcore_axis_name="s"),
        compiler_params=pltpu.CompilerParams(
            needs_layout_passes=False, use_tc_tiling_on_sc=True
        ),
        scratch_types=[
            pltpu.VMEM((C,), jnp.int32),
            pltpu.VMEM((C, W), jnp.float32),
            pltpu.VMEM((C, W), jnp.float32),
        ],
    )(x3, permutation)


def kernel(x, permutation):
    x3 = jnp.transpose(x, (0, 1, 3, 2)).reshape(G, C, W)
    out3 = _permute(x3, permutation)
    return jnp.transpose(out3.reshape(B, H, C, W), (0, 1, 3, 2))
